# baseline (speedup 1.0000x reference)
import functools

import jax
import jax.numpy as jnp
from jax import lax
from jax.experimental import pallas as pl
from jax.experimental.pallas import tpu as pltpu
from jax.experimental.pallas import tpu_sc as plsc

WINDOW = 8
NC = 2
NS = 16
LANES = 16


def _splat(ref, i):
    return plsc.load_gather(ref, [jnp.full((LANES,), i, jnp.int32)])


def _make_sc_call(B, L, C, H, W):
    HW = H * W
    P = min(512, HW)
    G = min(128, P)
    CAP = 4 * P + 2 * LANES
    NSTREAM = (4 * P) // G
    mesh = plsc.VectorSubcoreMesh(core_axis_name="c", subcore_axis_name="s",
                                  num_cores=NC, num_subcores=NS)

    @functools.partial(
        pl.kernel,
        out_type=jax.ShapeDtypeStruct((B * L, HW, C), jnp.float32),
        mesh=mesh,
        compiler_params=pltpu.CompilerParams(
            needs_layout_passes=False, use_tc_tiling_on_sc=False),
        scratch_types=[
            pltpu.VMEM((P,), jnp.float32),
            pltpu.VMEM((P,), jnp.float32),
            pltpu.VMEM((WINDOW, 2, P), jnp.float32),
            pltpu.VMEM((2 * CAP,), jnp.int32),
            pltpu.VMEM((2 * CAP,), jnp.float32),
            pltpu.VMEM((2 * CAP,), jnp.int32),
            pltpu.VMEM((4 * P, C), jnp.float32),
            pltpu.VMEM((P, C), jnp.float32),
            pltpu.VMEM((LANES,), jnp.float32),
            pltpu.SemaphoreType.DMA,
        ],
    )
    def sc_call(cpb_hbm, cum_hbm, img_hbm, wk_hbm, out_hbm,
                glx, gly, ckb, idxb, wcb, pcb, rowsb, acc, wkv, sem):
        cid = lax.axis_index("c")
        sid = lax.axis_index("s")
        f = sid * NC + cid
        b = f // L
        l = f % L
        k0 = jnp.maximum(l - (WINDOW - 1), 0)
        pltpu.sync_copy(wk_hbm, wkv)
        iota16 = lax.iota(jnp.int32, LANES)
        zero16i = jnp.zeros((LANES,), jnp.int32)
        zero16f = jnp.zeros((LANES,), jnp.float32)

        def zidx_body(z, _):
            z0 = pl.multiple_of(z * LANES, LANES)
            idxb[pl.ds(z0, LANES)] = zero16i
            return 0
        lax.fori_loop(0, (2 * CAP) // LANES, zidx_body, 0)

        def chunk_body(ci, _):
            c0 = pl.multiple_of(ci * P, P)
            pltpu.sync_copy(cpb_hbm.at[f, 0, pl.ds(c0, P)], glx)
            pltpu.sync_copy(cpb_hbm.at[f, 1, pl.ds(c0, P)], gly)
            pltpu.sync_copy(
                cum_hbm.at[pl.ds(b * L + k0, WINDOW), :, pl.ds(c0, P)], ckb)
            pltpu.sync_copy(img_hbm.at[pl.ds(f * HW + c0, P), :], acc)

            def compute_lists(kk, base):
                k = k0 + kk
                bk = b * L + k
                wkd = _splat(wkv, l - k)
                rowbase = bk * HW
                n = jnp.int32(0)
                for g in range(P // LANES):
                    s = pl.ds(g * LANES, LANES)
                    ix = glx[s] - ckb[kk, 0, s]
                    iy = gly[s] - ckb[kk, 1, s]
                    xt = ix.astype(jnp.int32)
                    x0 = xt - (xt.astype(jnp.float32) > ix).astype(jnp.int32)
                    fx = ix - x0.astype(jnp.float32)
                    yt = iy.astype(jnp.int32)
                    y0 = yt - (yt.astype(jnp.float32) > iy).astype(jnp.int32)
                    fy = iy - y0.astype(jnp.float32)
                    x1 = x0 + 1
                    y1 = y0 + 1
                    vx0 = x0.astype(jnp.uint32) < W
                    vx1 = x1.astype(jnp.uint32) < W
                    vy0 = y0.astype(jnp.uint32) < H
                    vy1 = y1.astype(jnp.uint32) < H
                    ofx = 1.0 - fx
                    ofy = 1.0 - fy
                    piota = iota16 + g * LANES
                    rx0 = rowbase + y0 * W + x0
                    taps = (
                        (vx0 & vy0, wkd * (ofx * ofy), rx0),
                        (vx0 & vy1, wkd * (ofx * fy), rx0 + W),
                        (vx1 & vy0, wkd * (fx * ofy), rx0 + 1),
                        (vx1 & vy1, wkd * (fx * fy), rx0 + W + 1),
                    )
                    for mask, wt, it in taps:
                        plsc.store_compressed(idxb.at[pl.ds(base + n, LANES)],
                                              it, mask=mask)
                        plsc.store_compressed(wcb.at[pl.ds(base + n, LANES)],
                                              wt, mask=mask)
                        plsc.store_compressed(pcb.at[pl.ds(base + n, LANES)],
                                              piota, mask=mask)
                        n = n + plsc.all_reduce_population_count(mask)[0]
                wcb[pl.ds(base + n, LANES)] = zero16f
                pcb[pl.ds(base + n, LANES)] = zero16i
                return n

            def fire(base, n):
                cps = []
                for j in range(NSTREAM):
                    off = pl.multiple_of(base + j * G, LANES)
                    sref = img_hbm.at[idxb.at[pl.ds(off, G)]]
                    dref = rowsb.at[pl.ds(j * G, G), :]

                    @pl.when(j * G < n)
                    def _(sref=sref, dref=dref):
                        cps.append(pltpu.async_copy(sref, dref, sem))
                return cps

            def drain(cps, n):
                for j in range(NSTREAM):
                    @pl.when(j * G < n)
                    def _(j=j):
                        cps[0].wait()
                        del cps[0]

            def accumulate(base, n):
                def blk_body(blk, _):
                    i0 = base + blk * LANES
                    wv16 = wcb[pl.ds(i0, LANES)]
                    pv16 = pcb[pl.ds(i0, LANES)]
                    for r in range(LANES):
                        i = blk * LANES + r
                        wv = jnp.full((LANES,), wv16[r])
                        p = pv16[r]
                        plsc.addupdate(acc.at[p, pl.ds(0, LANES)],
                                       wv * rowsb[i, pl.ds(0, LANES)])
                        plsc.addupdate(acc.at[p, pl.ds(LANES, LANES)],
                                       wv * rowsb[i, pl.ds(LANES, LANES)])
                    return 0
                nblk = (n + LANES - 1) // LANES
                lax.fori_loop(0, nblk, blk_body, 0)

            nk = l - k0

            @pl.when(nk > 0)
            def _():
                n0 = compute_lists(0, 0)

                def k_body(kk, n_prev):
                    pbase = pl.multiple_of((1 - (kk % 2)) * CAP, LANES)
                    cbase = pl.multiple_of((kk % 2) * CAP, LANES)
                    cps = fire(pbase, n_prev)
                    n_cur = compute_lists(kk, cbase)
                    drain(cps, n_prev)
                    accumulate(pbase, n_prev)
                    return n_cur

                n_last = lax.fori_loop(1, nk, k_body, n0)
                lbase = pl.multiple_of(((nk - 1) % 2) * CAP, LANES)
                cps = fire(lbase, n_last)
                drain(cps, n_last)
                accumulate(lbase, n_last)
            pltpu.sync_copy(acc, out_hbm.at[f, pl.ds(c0, P), :])
            return 0

        lax.fori_loop(0, HW // P, chunk_body, 0)

    return sc_call


def kernel(flows, images, decay_log):
    B, L, C, H, W = images.shape
    HW = H * W
    cum = jnp.cumsum(flows.astype(jnp.float32), axis=1)
    gx = jnp.linspace(-1.0 + 1.0 / W, 1.0 - 1.0 / W, W)
    gy = jnp.linspace(-1.0 + 1.0 / H, 1.0 - 1.0 / H, H)
    mx, my = jnp.meshgrid(gx, gy, indexing="xy")
    base = jnp.stack([mx, my], axis=0).astype(jnp.float32)
    scale = jnp.array([W * 0.5, H * 0.5], jnp.float32).reshape(1, 1, 2, 1, 1)
    av = (cum + base[None, None] + 1.0) * scale - 0.5
    bv = cum * scale
    cpb2 = av.reshape(B * L, 2, HW)
    cum2 = bv.reshape(B * L, 2, HW)
    imgflat = (images.astype(jnp.float32)
               .transpose(0, 1, 3, 4, 2)
               .reshape(B * L * HW, C))
    decay = jnp.exp(decay_log)
    dist = jnp.arange(LANES, dtype=jnp.float32)
    wks = jnp.exp(-decay * dist)
    out = _make_sc_call(B, L, C, H, W)(cpb2, cum2, imgflat, wks)
    out = out.reshape(B, L, H, W, C).transpose(0, 1, 4, 2, 3)
    return out.astype(images.dtype)


# speedup vs baseline: 1.0717x; 1.0717x over previous
import functools

import jax
import jax.numpy as jnp
from jax import lax
from jax.experimental import pallas as pl
from jax.experimental.pallas import tpu as pltpu
from jax.experimental.pallas import tpu_sc as plsc

WINDOW = 8
NC = 2
NS = 16
LANES = 16


def _splat(ref, i):
    return plsc.load_gather(ref, [jnp.full((LANES,), i, jnp.int32)])


def _make_sc_call(B, L, C, H, W):
    HW = H * W
    P = min(512, HW)
    G = min(128, P)
    CAP = 4 * P + 2 * LANES
    NSTREAM = (4 * P) // G
    NCHUNK = HW // P
    MAXI = 32
    mesh = plsc.VectorSubcoreMesh(core_axis_name="c", subcore_axis_name="s",
                                  num_cores=NC, num_subcores=NS)

    @functools.partial(
        pl.kernel,
        out_type=jax.ShapeDtypeStruct((B * L, HW, C), jnp.float32),
        mesh=mesh,
        compiler_params=pltpu.CompilerParams(
            needs_layout_passes=False, use_tc_tiling_on_sc=False),
        scratch_types=[
            pltpu.VMEM((P,), jnp.float32),
            pltpu.VMEM((P,), jnp.float32),
            pltpu.VMEM((WINDOW, 2, P), jnp.float32),
            pltpu.VMEM((CAP,), jnp.int32),
            pltpu.VMEM((CAP,), jnp.float32),
            pltpu.VMEM((CAP,), jnp.int32),
            pltpu.VMEM((4 * P, C), jnp.float32),
            pltpu.VMEM((P, C), jnp.float32),
            pltpu.VMEM((LANES,), jnp.float32),
            pltpu.VMEM((MAXI + LANES,), jnp.int32),
            pltpu.SemaphoreType.DMA,
        ],
    )
    def sc_call(cpb_hbm, cum_hbm, img_hbm, wk_hbm, sched_hbm, out_hbm,
                glx, gly, ckb, idxb, wcb, pcb, rowsb, acc, wkv, schv, sem):
        cid = lax.axis_index("c")
        sid = lax.axis_index("s")
        wid = sid * NC + cid
        pltpu.sync_copy(wk_hbm, wkv)
        pltpu.sync_copy(sched_hbm.at[wid], schv)
        iota16 = lax.iota(jnp.int32, LANES)
        zero16i = jnp.zeros((LANES,), jnp.int32)
        zero16f = jnp.zeros((LANES,), jnp.float32)

        def zidx_body(z, _):
            z0 = pl.multiple_of(z * LANES, LANES)
            idxb[pl.ds(z0, LANES)] = zero16i
            return 0
        lax.fori_loop(0, CAP // LANES, zidx_body, 0)

        def item_body(it, _):
            item = schv[pl.ds(it, LANES)][0]
            f = item // NCHUNK
            ci = item % NCHUNK
            b = f // L
            l = f % L
            k0 = jnp.maximum(l - (WINDOW - 1), 0)
            c0 = pl.multiple_of(ci * P, P)
            pltpu.sync_copy(cpb_hbm.at[f, 0, pl.ds(c0, P)], glx)
            pltpu.sync_copy(cpb_hbm.at[f, 1, pl.ds(c0, P)], gly)
            pltpu.sync_copy(
                cum_hbm.at[pl.ds(b * L + k0, WINDOW), :, pl.ds(c0, P)], ckb)
            pltpu.sync_copy(img_hbm.at[pl.ds(f * HW + c0, P), :], acc)

            def k_body(k, _):
                bk = b * L + k
                kk = k - k0
                wkd = _splat(wkv, l - k)
                rowbase = bk * HW
                n = jnp.int32(0)

                for g in range(P // LANES):
                    s = pl.ds(g * LANES, LANES)
                    ix = glx[s] - ckb[kk, 0, s]
                    iy = gly[s] - ckb[kk, 1, s]
                    xt = ix.astype(jnp.int32)
                    x0 = xt - (xt.astype(jnp.float32) > ix).astype(jnp.int32)
                    fx = ix - x0.astype(jnp.float32)
                    yt = iy.astype(jnp.int32)
                    y0 = yt - (yt.astype(jnp.float32) > iy).astype(jnp.int32)
                    fy = iy - y0.astype(jnp.float32)
                    x1 = x0 + 1
                    y1 = y0 + 1
                    vx0 = x0.astype(jnp.uint32) < W
                    vx1 = x1.astype(jnp.uint32) < W
                    vy0 = y0.astype(jnp.uint32) < H
                    vy1 = y1.astype(jnp.uint32) < H
                    ofx = 1.0 - fx
                    ofy = 1.0 - fy
                    piota = iota16 + g * LANES
                    rx0 = rowbase + y0 * W + x0
                    taps = (
                        (vx0 & vy0, wkd * (ofx * ofy), rx0),
                        (vx0 & vy1, wkd * (ofx * fy), rx0 + W),
                        (vx1 & vy0, wkd * (fx * ofy), rx0 + 1),
                        (vx1 & vy1, wkd * (fx * fy), rx0 + W + 1),
                    )
                    for mask, wt, srow in taps:
                        plsc.store_compressed(idxb.at[pl.ds(n, LANES)],
                                              srow, mask=mask)
                        plsc.store_compressed(wcb.at[pl.ds(n, LANES)],
                                              wt, mask=mask)
                        plsc.store_compressed(pcb.at[pl.ds(n, LANES)],
                                              piota, mask=mask)
                        n = n + plsc.all_reduce_population_count(mask)[0]

                wcb[pl.ds(n, LANES)] = zero16f
                pcb[pl.ds(n, LANES)] = zero16i

                cps = []
                for j in range(NSTREAM):
                    @pl.when(j * G < n)
                    def _(j=j):
                        cps.append(pltpu.async_copy(
                            img_hbm.at[idxb.at[pl.ds(j * G, G)]],
                            rowsb.at[pl.ds(j * G, G), :], sem))
                for j in range(NSTREAM):
                    @pl.when(j * G < n)
                    def _(j=j):
                        cps[0].wait()
                        del cps[0]

                def blk_body(blk, _):
                    i0 = blk * LANES
                    wv16 = wcb[pl.ds(i0, LANES)]
                    pv16 = pcb[pl.ds(i0, LANES)]
                    for r in range(LANES):
                        i = i0 + r
                        wv = jnp.full((LANES,), wv16[r])
                        p = pv16[r]
                        plsc.addupdate(acc.at[p, pl.ds(0, LANES)],
                                       wv * rowsb[i, pl.ds(0, LANES)])
                        plsc.addupdate(acc.at[p, pl.ds(LANES, LANES)],
                                       wv * rowsb[i, pl.ds(LANES, LANES)])
                    return 0
                nblk = (n + LANES - 1) // LANES
                lax.fori_loop(0, nblk, blk_body, 0)
                return 0

            lax.fori_loop(k0, l, k_body, 0)
            pltpu.sync_copy(acc, out_hbm.at[f, pl.ds(c0, P), :])
            return 0

        nit = schv[pl.ds(MAXI, LANES)][0]
        lax.fori_loop(0, nit, item_body, 0)

    return sc_call


def _build_schedule(B, L, HW):
    P = min(512, HW)
    nchunk = HW // P
    maxi = 32
    nw = NC * NS
    items = []
    for f in range(B * L):
        l = f % L
        cost = min(l, WINDOW - 1) + 0.35
        for ci in range(nchunk):
            items.append((cost, f * nchunk + ci))
    items.sort(key=lambda x: -x[0])
    loads = [0.0] * nw
    lists = [[] for _ in range(nw)]
    for cost, code in items:
        w = min(range(nw), key=lambda i: (loads[i], len(lists[i])))
        if len(lists[w]) >= maxi:
            w = min(range(nw), key=lambda i: len(lists[i]))
        lists[w].append(code)
        loads[w] += cost
    sched = [[0] * (maxi + LANES) for _ in range(nw)]
    for w in range(nw):
        for j, code in enumerate(lists[w]):
            sched[w][j] = code
        sched[w][maxi] = len(lists[w])
    return jnp.array(sched, dtype=jnp.int32)


def kernel(flows, images, decay_log):
    B, L, C, H, W = images.shape
    HW = H * W
    cum = jnp.cumsum(flows.astype(jnp.float32), axis=1)
    gx = jnp.linspace(-1.0 + 1.0 / W, 1.0 - 1.0 / W, W)
    gy = jnp.linspace(-1.0 + 1.0 / H, 1.0 - 1.0 / H, H)
    mx, my = jnp.meshgrid(gx, gy, indexing="xy")
    base = jnp.stack([mx, my], axis=0).astype(jnp.float32)
    scale = jnp.array([W * 0.5, H * 0.5], jnp.float32).reshape(1, 1, 2, 1, 1)
    av = (cum + base[None, None] + 1.0) * scale - 0.5
    bv = cum * scale
    cpb2 = av.reshape(B * L, 2, HW)
    cum2 = bv.reshape(B * L, 2, HW)
    imgflat = (images.astype(jnp.float32)
               .transpose(0, 1, 3, 4, 2)
               .reshape(B * L * HW, C))
    decay = jnp.exp(decay_log)
    dist = jnp.arange(LANES, dtype=jnp.float32)
    wks = jnp.exp(-decay * dist)
    sched = _build_schedule(B, L, HW)
    out = _make_sc_call(B, L, C, H, W)(cpb2, cum2, imgflat, wks, sched)
    out = out.reshape(B, L, H, W, C).transpose(0, 1, 4, 2, 3)
    return out.astype(images.dtype)
